# baseline (speedup 1.0000x reference)
import jax
import jax.numpy as jnp
from jax.experimental import pallas as pl

NUM_GROUP = 256
GROUP_SIZE = 32


def _fps(xyz, npoint):
    B, N, C = xyz.shape
    barycenter = jnp.mean(xyz, axis=1, keepdims=True)
    dist = jnp.sum((xyz - barycenter) ** 2, axis=-1)
    farthest = jnp.argmax(dist, axis=1)
    distance = jnp.full((B, N), 1e10, dtype=xyz.dtype)
    centroids = jnp.zeros((B, npoint), dtype=farthest.dtype)
    batch_idx = jnp.arange(B)

    def body(i, carry):
        centroids, distance, farthest = carry
        centroids = centroids.at[:, i].set(farthest)
        centroid = xyz[batch_idx, farthest][:, None, :]
        d = jnp.sum((xyz - centroid) ** 2, axis=-1)
        distance = jnp.minimum(distance, d)
        farthest = jnp.argmax(distance, axis=-1)
        return (centroids, distance, farthest)

    centroids, _, _ = jax.lax.fori_loop(0, npoint, body, (centroids, distance, farthest))
    return centroids


def _sub_kernel(n_ref, c_ref, o_ref):
    o_ref[...] = n_ref[...] - c_ref[...]


def kernel(xyz):
    B, N, C = xyz.shape
    batch_idx = jnp.arange(B)[:, None]
    center_index = _fps(jax.lax.stop_gradient(xyz), NUM_GROUP)
    center = xyz[batch_idx, center_index, :]
    d_sq = (jnp.sum(center ** 2, axis=-1)[:, :, None]
            + jnp.sum(xyz ** 2, axis=-1)[:, None, :]
            - 2.0 * jnp.einsum('bgd,bnd->bgn', center, xyz))
    _, idx = jax.lax.top_k(-d_sq, GROUP_SIZE)
    idx_base = jnp.arange(B)[:, None, None] * N
    idx_flat = (idx + idx_base).reshape(-1)
    flat = xyz.reshape(B * N, -1)
    gathered = flat[idx_flat, :].reshape(B, NUM_GROUP, GROUP_SIZE, C)
    cbc = jnp.broadcast_to(center[:, :, None, :], gathered.shape)
    g2 = gathered.reshape(B * NUM_GROUP, GROUP_SIZE * C)
    c2 = cbc.reshape(B * NUM_GROUP, GROUP_SIZE * C)
    neighborhood = pl.pallas_call(
        _sub_kernel,
        out_shape=jax.ShapeDtypeStruct(g2.shape, g2.dtype),
    )(g2, c2)
    return (neighborhood.reshape(B, NUM_GROUP, GROUP_SIZE, C), center)


# speedup vs baseline: 1.2980x; 1.2980x over previous
import functools

import jax
import jax.numpy as jnp
from jax import lax
from jax.experimental import pallas as pl
from jax.experimental.pallas import tpu as pltpu
from jax.experimental.pallas import tpu_sc as plsc

NUM_GROUP = 256
GROUP_SIZE = 32
B = 8
N = 16384
NVREG_N = N // 16
BIGI = 2147483647


def _fps_body(xp, yp, zp, out, xl, yl, zl, dist, cstage):
    c = lax.axis_index("c")
    s = lax.axis_index("s")
    b = c * 4 + s
    iota = lax.iota(jnp.int32, 16)
    lane0 = iota == 0

    @pl.when(s < 4)
    def _():
        pltpu.sync_copy(xp.at[b], xl)
        pltpu.sync_copy(yp.at[b], yl)
        pltpu.sync_copy(zp.at[b], zl)

        def initbody(j, carry):
            dist[pl.ds(j * 16, 16)] = jnp.full((16,), 1e10, jnp.float32)
            return carry
        lax.fori_loop(0, NVREG_N, initbody, 0)

        z16 = jnp.zeros((16,), jnp.float32)

        def sumbody(j, carry):
            sx, sy, sz = carry
            off = j * 16
            return (sx + xl[pl.ds(off, 16)],
                    sy + yl[pl.ds(off, 16)],
                    sz + zl[pl.ds(off, 16)])
        sx, sy, sz = lax.fori_loop(0, NVREG_N, sumbody, (z16, z16, z16))
        bx = jnp.sum(sx) * (1.0 / N)
        by = jnp.sum(sy) * (1.0 / N)
        bz = jnp.sum(sz) * (1.0 / N)

        def pick(m, mi):
            mval = jnp.max(m)
            li = jnp.min(jnp.where(m == mval, mi, BIGI))
            liv = li + jnp.zeros((16,), jnp.int32)
            cx = plsc.load_gather(xl, [liv])
            cy = plsc.load_gather(yl, [liv])
            cz = plsc.load_gather(zl, [liv])
            return cx, cy, cz

        def bmax(j, carry):
            m, mi = carry
            off = j * 16
            dx = xl[pl.ds(off, 16)] - bx
            dy = yl[pl.ds(off, 16)] - by
            dz = zl[pl.ds(off, 16)] - bz
            d = dx * dx + dy * dy + dz * dz
            cmp = d > m
            m = jnp.where(cmp, d, m)
            mi = jnp.where(cmp, off + iota, mi)
            return (m, mi)
        m, mi = lax.fori_loop(0, NVREG_N, bmax,
                              (jnp.full((16,), -1.0, jnp.float32),
                               jnp.zeros((16,), jnp.int32)))
        cx, cy, cz = pick(m, mi)

        def mbody(i, carry):
            cx, cy, cz = carry
            base = 3 * i + jnp.zeros((16,), jnp.int32)
            plsc.store_scatter(cstage, [base], cx, mask=lane0)
            plsc.store_scatter(cstage, [base + 1], cy, mask=lane0)
            plsc.store_scatter(cstage, [base + 2], cz, mask=lane0)

            def dbody(j, carry2):
                m, mi = carry2
                off = j * 16
                dx = xl[pl.ds(off, 16)] - cx
                dy = yl[pl.ds(off, 16)] - cy
                dz = zl[pl.ds(off, 16)] - cz
                d = dx * dx + dy * dy + dz * dz
                dd = jnp.minimum(dist[pl.ds(off, 16)], d)
                dist[pl.ds(off, 16)] = dd
                cmp = dd > m
                m = jnp.where(cmp, dd, m)
                mi = jnp.where(cmp, off + iota, mi)
                return (m, mi)
            m2, mi2 = lax.fori_loop(0, NVREG_N, dbody,
                                    (jnp.full((16,), -1.0, jnp.float32),
                                     jnp.zeros((16,), jnp.int32)))
            return pick(m2, mi2)

        lax.fori_loop(0, NUM_GROUP, mbody, (cx, cy, cz))
        pltpu.sync_copy(cstage, out.at[pl.ds(b * (NUM_GROUP * 3), NUM_GROUP * 3)])


@jax.jit
def _fps_centers(xp, yp, zp):
    mesh = plsc.VectorSubcoreMesh(core_axis_name="c", subcore_axis_name="s")
    f = functools.partial(
        pl.kernel,
        mesh=mesh,
        compiler_params=pltpu.CompilerParams(needs_layout_passes=False),
        out_type=jax.ShapeDtypeStruct((B * NUM_GROUP * 3,), jnp.float32),
        scratch_types=[
            pltpu.VMEM((N,), jnp.float32),
            pltpu.VMEM((N,), jnp.float32),
            pltpu.VMEM((N,), jnp.float32),
            pltpu.VMEM((N,), jnp.float32),
            pltpu.VMEM((NUM_GROUP * 3,), jnp.float32),
        ],
    )(_fps_body)
    return f(xp, yp, zp)


def kernel(xyz):
    planes = jnp.transpose(xyz, (2, 0, 1))
    centers_flat = _fps_centers(planes[0], planes[1], planes[2])
    center = centers_flat.reshape(B, NUM_GROUP, 3)

    d_sq = (jnp.sum(center ** 2, axis=-1)[:, :, None]
            + jnp.sum(xyz ** 2, axis=-1)[:, None, :]
            - 2.0 * jnp.einsum('bgd,bnd->bgn', center, xyz))
    _, idx = jax.lax.top_k(-d_sq, GROUP_SIZE)
    idx_base = jnp.arange(B)[:, None, None] * N
    idx_flat = (idx + idx_base).reshape(-1)
    flat = xyz.reshape(B * N, -1)
    neighborhood = flat[idx_flat, :].reshape(B, NUM_GROUP, GROUP_SIZE, 3)
    neighborhood = neighborhood - center[:, :, None, :]
    return (neighborhood, center)


# speedup vs baseline: 3.0335x; 2.3370x over previous
import functools

import jax
import jax.numpy as jnp
from jax import lax
from jax.experimental import pallas as pl
from jax.experimental.pallas import tpu as pltpu
from jax.experimental.pallas import tpu_sc as plsc

NUM_GROUP = 256
GROUP_SIZE = 32
B = 8
N = 16384
NVREG_N = N // 16
BIGI = 2147483647


def _fps_body(xp, yp, zp, out, xl, yl, zl, dist, cstage):
    c = lax.axis_index("c")
    s = lax.axis_index("s")
    b = c * 4 + s
    iota = lax.iota(jnp.int32, 16)
    lane0 = iota == 0

    @pl.when(s < 4)
    def _():
        pltpu.sync_copy(xp.at[b], xl)
        pltpu.sync_copy(yp.at[b], yl)
        pltpu.sync_copy(zp.at[b], zl)

        def initbody(j, carry):
            dist[pl.ds(j * 16, 16)] = jnp.full((16,), 1e10, jnp.float32)
            return carry
        lax.fori_loop(0, NVREG_N, initbody, 0)

        z16 = jnp.zeros((16,), jnp.float32)

        def sumbody(j, carry):
            sx, sy, sz = carry
            off = j * 16
            return (sx + xl[pl.ds(off, 16)],
                    sy + yl[pl.ds(off, 16)],
                    sz + zl[pl.ds(off, 16)])
        sx, sy, sz = lax.fori_loop(0, NVREG_N, sumbody, (z16, z16, z16))
        bx = jnp.sum(sx) * (1.0 / N)
        by = jnp.sum(sy) * (1.0 / N)
        bz = jnp.sum(sz) * (1.0 / N)

        def pick(m, mi):
            mval = jnp.max(m)
            li = jnp.min(jnp.where(m == mval, mi, BIGI))
            liv = li + jnp.zeros((16,), jnp.int32)
            cx = plsc.load_gather(xl, [liv])
            cy = plsc.load_gather(yl, [liv])
            cz = plsc.load_gather(zl, [liv])
            return cx, cy, cz

        def bmax(j, carry):
            m, mi = carry
            off = j * 16
            dx = xl[pl.ds(off, 16)] - bx
            dy = yl[pl.ds(off, 16)] - by
            dz = zl[pl.ds(off, 16)] - bz
            d = dx * dx + dy * dy + dz * dz
            cmp = d > m
            m = jnp.where(cmp, d, m)
            mi = jnp.where(cmp, off + iota, mi)
            return (m, mi)
        m, mi = lax.fori_loop(0, NVREG_N, bmax,
                              (jnp.full((16,), -1.0, jnp.float32),
                               jnp.zeros((16,), jnp.int32)))
        cx, cy, cz = pick(m, mi)

        def mbody(i, carry):
            cx, cy, cz = carry
            base = 3 * i + jnp.zeros((16,), jnp.int32)
            plsc.store_scatter(cstage, [base], cx, mask=lane0)
            plsc.store_scatter(cstage, [base + 1], cy, mask=lane0)
            plsc.store_scatter(cstage, [base + 2], cz, mask=lane0)

            def dbody(j, carry2):
                m, mi = carry2
                off = j * 16
                dx = xl[pl.ds(off, 16)] - cx
                dy = yl[pl.ds(off, 16)] - cy
                dz = zl[pl.ds(off, 16)] - cz
                d = dx * dx + dy * dy + dz * dz
                dd = jnp.minimum(dist[pl.ds(off, 16)], d)
                dist[pl.ds(off, 16)] = dd
                cmp = dd > m
                m = jnp.where(cmp, dd, m)
                mi = jnp.where(cmp, off + iota, mi)
                return (m, mi)
            m2, mi2 = lax.fori_loop(0, NVREG_N, dbody,
                                    (jnp.full((16,), -1.0, jnp.float32),
                                     jnp.zeros((16,), jnp.int32)))
            return pick(m2, mi2)

        lax.fori_loop(0, NUM_GROUP, mbody, (cx, cy, cz))
        pltpu.sync_copy(cstage, out.at[pl.ds(b * (NUM_GROUP * 3), NUM_GROUP * 3)])


@jax.jit
def _fps_centers(xp, yp, zp):
    mesh = plsc.VectorSubcoreMesh(core_axis_name="c", subcore_axis_name="s")
    f = functools.partial(
        pl.kernel,
        mesh=mesh,
        compiler_params=pltpu.CompilerParams(needs_layout_passes=False),
        out_type=jax.ShapeDtypeStruct((B * NUM_GROUP * 3,), jnp.float32),
        scratch_types=[
            pltpu.VMEM((N,), jnp.float32),
            pltpu.VMEM((N,), jnp.float32),
            pltpu.VMEM((N,), jnp.float32),
            pltpu.VMEM((N,), jnp.float32),
            pltpu.VMEM((NUM_GROUP * 3,), jnp.float32),
        ],
    )(_fps_body)
    return f(xp, yp, zp)


ROWS_PER_W = (B * NUM_GROUP) // 32
BUFCAP = 144
INF = float("inf")


def _bfr16(v):
    u = plsc.bitcast(v, jnp.uint32)
    r = (u + jnp.uint32(0x7FFF) + ((u >> 16) & jnp.uint32(1))) \
        & jnp.uint32(0xFFFF0000)
    return plsc.bitcast(r, jnp.float32)


def _knn_body(xp, yp, zp, centers, out, xl, yl, zl, sq, xb, yb, zb, cloc,
              dbuf, ibuf, t32d, t32i, ostage):
    c = lax.axis_index("c")
    s = lax.axis_index("s")
    w = c * 16 + s
    b = w // 4
    g0 = (w % 4) * ROWS_PER_W
    iota = lax.iota(jnp.int32, 16)

    pltpu.sync_copy(xp.at[b], xl)
    pltpu.sync_copy(yp.at[b], yl)
    pltpu.sync_copy(zp.at[b], zl)
    pltpu.sync_copy(centers.at[pl.ds(b * (NUM_GROUP * 3) + g0 * 3,
                                     ROWS_PER_W * 3)],
                    cloc.at[pl.ds(0, ROWS_PER_W * 3)])

    def sqbody(j, carry):
        off = j * 16
        x = xl[pl.ds(off, 16)]
        y = yl[pl.ds(off, 16)]
        z = zl[pl.ds(off, 16)]
        sq[pl.ds(off, 16)] = x * x + y * y + z * z
        xb[pl.ds(off, 16)] = _bfr16(x)
        yb[pl.ds(off, 16)] = _bfr16(y)
        zb[pl.ds(off, 16)] = _bfr16(z)
        return carry
    lax.fori_loop(0, NVREG_N, sqbody, 0)

    def merge16(lo_d, lo_i, hi_d, hi_i, nd, ni):
        rd = lax.rev(nd, (0,))
        ri = lax.rev(ni, (0,))
        sel = lo_d <= rd
        a_d = jnp.where(sel, lo_d, rd)
        a_i = jnp.where(sel, lo_i, ri)
        s_d = jnp.where(sel, rd, lo_d)
        s_i = jnp.where(sel, ri, lo_i)
        a_d, a_i = plsc.sort_key_val(a_d, a_i)
        s_d, s_i = plsc.sort_key_val(s_d, s_i)
        rs_d = lax.rev(s_d, (0,))
        rs_i = lax.rev(s_i, (0,))
        sel2 = hi_d <= rs_d
        c_d = jnp.where(sel2, hi_d, rs_d)
        c_i = jnp.where(sel2, hi_i, rs_i)
        c_d, c_i = plsc.sort_key_val(c_d, c_i)
        return a_d, a_i, c_d, c_i

    def compact(pos):
        lo_d = t32d[pl.ds(0, 16)]
        lo_i = t32i[pl.ds(0, 16)]
        hi_d = t32d[pl.ds(16, 16)]
        hi_i = t32i[pl.ds(16, 16)]
        nb = (pos + 15) // 16

        def cbody(k, carry):
            lo_d, lo_i, hi_d, hi_i = carry
            off = k * 16
            valid = (off + iota) < pos
            nd = jnp.where(valid, dbuf[pl.ds(off, 16)], INF)
            ni = ibuf[pl.ds(off, 16)]
            nd, ni = plsc.sort_key_val(nd, ni)
            return merge16(lo_d, lo_i, hi_d, hi_i, nd, ni)
        lo_d, lo_i, hi_d, hi_i = lax.fori_loop(
            0, nb, cbody, (lo_d, lo_i, hi_d, hi_i))
        t32d[pl.ds(0, 16)] = lo_d
        t32i[pl.ds(0, 16)] = lo_i
        t32d[pl.ds(16, 16)] = hi_d
        t32i[pl.ds(16, 16)] = hi_i

    def row(j, carry):
        cv = cloc[pl.ds(3 * j, 16)]
        cvb = _bfr16(cv)
        cx = cv[0]
        cy = cv[1]
        cz = cv[2]
        cxb = cvb[0]
        cyb = cvb[1]
        czb = cvb[2]
        cc = cx * cx + cy * cy + cz * cz
        t32d[pl.ds(0, 16)] = jnp.full((16,), INF, jnp.float32)
        t32d[pl.ds(16, 16)] = jnp.full((16,), INF, jnp.float32)
        t32i[pl.ds(0, 16)] = jnp.zeros((16,), jnp.int32)
        t32i[pl.ds(16, 16)] = jnp.zeros((16,), jnp.int32)

        def scan(j2, carry2):
            pos, t = carry2
            off = j2 * 16
            x = xb[pl.ds(off, 16)]
            y = yb[pl.ds(off, 16)]
            z = zb[pl.ds(off, 16)]
            sv = sq[pl.ds(off, 16)]
            tt = cxb * x + cyb * y + czb * z
            d = (cc + sv) - 2.0 * tt
            msk = d < t

            def hit(pos, t):
                plsc.store_compressed(dbuf.at[pl.ds(pos, 16)], d, mask=msk)
                plsc.store_compressed(ibuf.at[pl.ds(pos, 16)], off + iota,
                                      mask=msk)
                cnt = jnp.max(plsc.all_reduce_population_count(msk))
                pos2 = pos + cnt

                def do_compact(p, tt_):
                    compact(p)
                    return (jnp.int32(0), t32d[pl.ds(16, 16)][15])
                return lax.cond(pos2 >= 112, do_compact,
                                lambda p, tt_: (p, tt_), pos2, t)
            return lax.cond(jnp.any(msk), hit, lambda p, tt_: (p, tt_),
                            pos, t)

        pos, _ = lax.fori_loop(0, NVREG_N, scan,
                               (jnp.int32(0), jnp.float32(INF)))
        compact(pos)

        for part in range(2):
            iv = t32i[pl.ds(16 * part, 16)]
            nx = plsc.load_gather(xl, [iv]) - cx
            ny = plsc.load_gather(yl, [iv]) - cy
            nz = plsc.load_gather(zl, [iv]) - cz
            p3 = part * 48 + iota * 3
            plsc.store_scatter(ostage, [p3], nx)
            plsc.store_scatter(ostage, [p3 + 1], ny)
            plsc.store_scatter(ostage, [p3 + 2], nz)
        pltpu.sync_copy(
            ostage,
            out.at[pl.ds((b * NUM_GROUP + g0 + j) * (GROUP_SIZE * 3),
                         GROUP_SIZE * 3)])
        return carry

    lax.fori_loop(0, ROWS_PER_W, row, 0)


@jax.jit
def _knn_groups(xp, yp, zp, centers):
    mesh = plsc.VectorSubcoreMesh(core_axis_name="c", subcore_axis_name="s")
    f = functools.partial(
        pl.kernel,
        mesh=mesh,
        compiler_params=pltpu.CompilerParams(needs_layout_passes=False),
        out_type=jax.ShapeDtypeStruct((B * NUM_GROUP * GROUP_SIZE * 3,),
                                      jnp.float32),
        scratch_types=[
            pltpu.VMEM((N,), jnp.float32),
            pltpu.VMEM((N,), jnp.float32),
            pltpu.VMEM((N,), jnp.float32),
            pltpu.VMEM((N,), jnp.float32),
            pltpu.VMEM((N,), jnp.float32),
            pltpu.VMEM((N,), jnp.float32),
            pltpu.VMEM((N,), jnp.float32),
            pltpu.VMEM((ROWS_PER_W * 3 + 16,), jnp.float32),
            pltpu.VMEM((BUFCAP,), jnp.float32),
            pltpu.VMEM((BUFCAP,), jnp.int32),
            pltpu.VMEM((32,), jnp.float32),
            pltpu.VMEM((32,), jnp.int32),
            pltpu.VMEM((GROUP_SIZE * 3,), jnp.float32),
        ],
    )(_knn_body)
    return f(xp, yp, zp, centers)


def kernel(xyz):
    planes = jnp.transpose(xyz, (2, 0, 1))
    centers_flat = _fps_centers(planes[0], planes[1], planes[2])
    center = centers_flat.reshape(B, NUM_GROUP, 3)
    nb_flat = _knn_groups(planes[0], planes[1], planes[2], centers_flat)
    neighborhood = nb_flat.reshape(B, NUM_GROUP, GROUP_SIZE, 3)
    return (neighborhood, center)


# speedup vs baseline: 4.5532x; 1.5010x over previous
import functools

import jax
import jax.numpy as jnp
from jax import lax
from jax.experimental import pallas as pl
from jax.experimental.pallas import tpu as pltpu
from jax.experimental.pallas import tpu_sc as plsc

NUM_GROUP = 256
GROUP_SIZE = 32
B = 8
N = 16384
NVREG_N = N // 16
BIGI = 2147483647


def _fps_body(xp, yp, zp, out, xl, yl, zl, dist, cstage):
    c = lax.axis_index("c")
    s = lax.axis_index("s")
    b = c * 4 + s
    iota = lax.iota(jnp.int32, 16)
    lane0 = iota == 0

    @pl.when(s < 4)
    def _():
        pltpu.sync_copy(xp.at[b], xl)
        pltpu.sync_copy(yp.at[b], yl)
        pltpu.sync_copy(zp.at[b], zl)

        def initbody(j, carry):
            dist[pl.ds(j * 16, 16)] = jnp.full((16,), 1e10, jnp.float32)
            return carry
        lax.fori_loop(0, NVREG_N, initbody, 0)

        z16 = jnp.zeros((16,), jnp.float32)

        def sumbody(j, carry):
            sx, sy, sz = carry
            off = j * 16
            return (sx + xl[pl.ds(off, 16)],
                    sy + yl[pl.ds(off, 16)],
                    sz + zl[pl.ds(off, 16)])
        sx, sy, sz = lax.fori_loop(0, NVREG_N, sumbody, (z16, z16, z16))
        bx = jnp.sum(sx) * (1.0 / N)
        by = jnp.sum(sy) * (1.0 / N)
        bz = jnp.sum(sz) * (1.0 / N)

        def pick(m, mi):
            mval = jnp.max(m)
            li = jnp.min(jnp.where(m == mval, mi, BIGI))
            liv = li + jnp.zeros((16,), jnp.int32)
            cx = plsc.load_gather(xl, [liv])
            cy = plsc.load_gather(yl, [liv])
            cz = plsc.load_gather(zl, [liv])
            return cx, cy, cz

        def bmax(j, carry):
            m, mi = carry
            off = j * 16
            dx = xl[pl.ds(off, 16)] - bx
            dy = yl[pl.ds(off, 16)] - by
            dz = zl[pl.ds(off, 16)] - bz
            d = dx * dx + dy * dy + dz * dz
            cmp = d > m
            m = jnp.where(cmp, d, m)
            mi = jnp.where(cmp, off + iota, mi)
            return (m, mi)
        m, mi = lax.fori_loop(0, NVREG_N, bmax,
                              (jnp.full((16,), -1.0, jnp.float32),
                               jnp.zeros((16,), jnp.int32)))
        cx, cy, cz = pick(m, mi)

        def mbody(i, carry):
            cx, cy, cz = carry
            base = 3 * i + jnp.zeros((16,), jnp.int32)
            plsc.store_scatter(cstage, [base], cx, mask=lane0)
            plsc.store_scatter(cstage, [base + 1], cy, mask=lane0)
            plsc.store_scatter(cstage, [base + 2], cz, mask=lane0)

            def dbody(j, carry2):
                m, mi = carry2
                for u in range(4):
                    off = (j * 4 + u) * 16
                    dx = xl[pl.ds(off, 16)] - cx
                    dy = yl[pl.ds(off, 16)] - cy
                    dz = zl[pl.ds(off, 16)] - cz
                    d = dx * dx + dy * dy + dz * dz
                    dd = jnp.minimum(dist[pl.ds(off, 16)], d)
                    dist[pl.ds(off, 16)] = dd
                    cmp = dd > m
                    m = jnp.where(cmp, dd, m)
                    mi = jnp.where(cmp, off + iota, mi)
                return (m, mi)
            m2, mi2 = lax.fori_loop(0, NVREG_N // 4, dbody,
                                    (jnp.full((16,), -1.0, jnp.float32),
                                     jnp.zeros((16,), jnp.int32)))
            return pick(m2, mi2)

        lax.fori_loop(0, NUM_GROUP, mbody, (cx, cy, cz))
        pltpu.sync_copy(cstage, out.at[pl.ds(b * (NUM_GROUP * 3), NUM_GROUP * 3)])


@jax.jit
def _fps_centers(xp, yp, zp):
    mesh = plsc.VectorSubcoreMesh(core_axis_name="c", subcore_axis_name="s")
    f = functools.partial(
        pl.kernel,
        mesh=mesh,
        compiler_params=pltpu.CompilerParams(needs_layout_passes=False),
        out_type=jax.ShapeDtypeStruct((B * NUM_GROUP * 3,), jnp.float32),
        scratch_types=[
            pltpu.VMEM((N,), jnp.float32),
            pltpu.VMEM((N,), jnp.float32),
            pltpu.VMEM((N,), jnp.float32),
            pltpu.VMEM((N,), jnp.float32),
            pltpu.VMEM((NUM_GROUP * 3,), jnp.float32),
        ],
    )(_fps_body)
    return f(xp, yp, zp)


ROWS_PER_W = (B * NUM_GROUP) // 32
BUFCAP = 144
INF = float("inf")


def _bfr16(v):
    u = plsc.bitcast(v, jnp.uint32)
    r = (u + jnp.uint32(0x7FFF) + ((u >> 16) & jnp.uint32(1))) \
        & jnp.uint32(0xFFFF0000)
    return plsc.bitcast(r, jnp.float32)


NCAND = 48


def _knn_body(xp, yp, zp, centers, outp, outi, xl, yl, zl, sq, xb, yb, zb,
              cloc, dbuf, ibuf, t48d, t48i, ostage):
    c = lax.axis_index("c")
    s = lax.axis_index("s")
    w = c * 16 + s
    b = w // 4
    g0 = (w % 4) * ROWS_PER_W
    iota = lax.iota(jnp.int32, 16)

    pltpu.sync_copy(xp.at[b], xl)
    pltpu.sync_copy(yp.at[b], yl)
    pltpu.sync_copy(zp.at[b], zl)
    pltpu.sync_copy(centers.at[pl.ds(b * (NUM_GROUP * 3) + g0 * 3,
                                     ROWS_PER_W * 3)],
                    cloc.at[pl.ds(0, ROWS_PER_W * 3)])

    def sqbody(j, carry):
        off = j * 16
        x = xl[pl.ds(off, 16)]
        y = yl[pl.ds(off, 16)]
        z = zl[pl.ds(off, 16)]
        sq[pl.ds(off, 16)] = x * x + y * y + z * z
        xb[pl.ds(off, 16)] = _bfr16(x)
        yb[pl.ds(off, 16)] = _bfr16(y)
        zb[pl.ds(off, 16)] = _bfr16(z)
        return carry
    lax.fori_loop(0, NVREG_N, sqbody, 0)

    def mergepair(a_d, a_i, b_d, b_i):
        rd = lax.rev(b_d, (0,))
        ri = lax.rev(b_i, (0,))
        sel = a_d <= rd
        lo_d = jnp.where(sel, a_d, rd)
        lo_i = jnp.where(sel, a_i, ri)
        hi_d = jnp.where(sel, rd, a_d)
        hi_i = jnp.where(sel, ri, a_i)
        lo_d, lo_i = plsc.sort_key_val(lo_d, lo_i)
        hi_d, hi_i = plsc.sort_key_val(hi_d, hi_i)
        return lo_d, lo_i, hi_d, hi_i

    def compact(pos):
        lo_d = t48d[pl.ds(0, 16)]
        lo_i = t48i[pl.ds(0, 16)]
        md_d = t48d[pl.ds(16, 16)]
        md_i = t48i[pl.ds(16, 16)]
        hi_d = t48d[pl.ds(32, 16)]
        hi_i = t48i[pl.ds(32, 16)]
        nb = (pos + 15) // 16

        def cbody(k, carry):
            lo_d, lo_i, md_d, md_i, hi_d, hi_i = carry
            off = k * 16
            valid = (off + iota) < pos
            nd = jnp.where(valid, dbuf[pl.ds(off, 16)], INF)
            ni = ibuf[pl.ds(off, 16)]
            nd, ni = plsc.sort_key_val(nd, ni)
            lo_d, lo_i, s_d, s_i = mergepair(lo_d, lo_i, nd, ni)
            md_d, md_i, s_d, s_i = mergepair(md_d, md_i, s_d, s_i)
            hi_d, hi_i, _, _ = mergepair(hi_d, hi_i, s_d, s_i)
            return (lo_d, lo_i, md_d, md_i, hi_d, hi_i)
        lo_d, lo_i, md_d, md_i, hi_d, hi_i = lax.fori_loop(
            0, nb, cbody, (lo_d, lo_i, md_d, md_i, hi_d, hi_i))
        t48d[pl.ds(0, 16)] = lo_d
        t48i[pl.ds(0, 16)] = lo_i
        t48d[pl.ds(16, 16)] = md_d
        t48i[pl.ds(16, 16)] = md_i
        t48d[pl.ds(32, 16)] = hi_d
        t48i[pl.ds(32, 16)] = hi_i

    def row(j, carry):
        cv = cloc[pl.ds(3 * j, 16)]
        cvb = _bfr16(cv)
        cx = cv[0]
        cy = cv[1]
        cz = cv[2]
        cxb = cvb[0]
        cyb = cvb[1]
        czb = cvb[2]
        cc = cx * cx + cy * cy + cz * cz
        for part in range(3):
            t48d[pl.ds(16 * part, 16)] = jnp.full((16,), INF, jnp.float32)
            t48i[pl.ds(16 * part, 16)] = jnp.zeros((16,), jnp.int32)

        def scan(j2, carry2):
            pos, t = carry2
            for u in range(4):
                off = (j2 * 4 + u) * 16
                x = xb[pl.ds(off, 16)]
                y = yb[pl.ds(off, 16)]
                z = zb[pl.ds(off, 16)]
                sv = sq[pl.ds(off, 16)]
                tt = cxb * x + cyb * y + czb * z
                d = (cc + sv) - 2.0 * tt
                msk = d < t
                plsc.store_compressed(dbuf.at[pl.ds(pos, 16)], d, mask=msk)
                plsc.store_compressed(ibuf.at[pl.ds(pos, 16)], off + iota,
                                      mask=msk)
                pos = pos + plsc.all_reduce_population_count(msk)[0]

            def do_compact(p, tt_):
                compact(p)
                return (jnp.int32(0), t48d[pl.ds(32, 16)][15])
            return lax.cond(pos >= 64, do_compact,
                            lambda p, tt_: (p, tt_), pos, t)

        pos, _ = lax.fori_loop(0, NVREG_N // 4, scan,
                               (jnp.int32(0), jnp.float32(INF)))
        compact(pos)

        for part in range(3):
            iv = t48i[pl.ds(16 * part, 16)]
            px = plsc.load_gather(xl, [iv])
            py = plsc.load_gather(yl, [iv])
            pz = plsc.load_gather(zl, [iv])
            p3 = part * 48 + iota * 3
            plsc.store_scatter(ostage, [p3], px)
            plsc.store_scatter(ostage, [p3 + 1], py)
            plsc.store_scatter(ostage, [p3 + 2], pz)
        row_id = b * NUM_GROUP + g0 + j
        pltpu.sync_copy(ostage, outp.at[pl.ds(row_id * (NCAND * 3),
                                              NCAND * 3)])
        pltpu.sync_copy(t48i, outi.at[pl.ds(row_id * NCAND, NCAND)])
        return carry

    lax.fori_loop(0, ROWS_PER_W, row, 0)


@jax.jit
def _knn_groups(xp, yp, zp, centers):
    mesh = plsc.VectorSubcoreMesh(core_axis_name="c", subcore_axis_name="s")
    f = functools.partial(
        pl.kernel,
        mesh=mesh,
        compiler_params=pltpu.CompilerParams(needs_layout_passes=False),
        out_type=(jax.ShapeDtypeStruct((B * NUM_GROUP * NCAND * 3,),
                                       jnp.float32),
                  jax.ShapeDtypeStruct((B * NUM_GROUP * NCAND,), jnp.int32)),
        scratch_types=[
            pltpu.VMEM((N,), jnp.float32),
            pltpu.VMEM((N,), jnp.float32),
            pltpu.VMEM((N,), jnp.float32),
            pltpu.VMEM((N,), jnp.float32),
            pltpu.VMEM((N,), jnp.float32),
            pltpu.VMEM((N,), jnp.float32),
            pltpu.VMEM((N,), jnp.float32),
            pltpu.VMEM((ROWS_PER_W * 3 + 16,), jnp.float32),
            pltpu.VMEM((BUFCAP,), jnp.float32),
            pltpu.VMEM((BUFCAP,), jnp.int32),
            pltpu.VMEM((NCAND,), jnp.float32),
            pltpu.VMEM((NCAND,), jnp.int32),
            pltpu.VMEM((NCAND * 3,), jnp.float32),
        ],
    )(_knn_body)
    return f(xp, yp, zp, centers)


def kernel(xyz):
    planes = jnp.transpose(xyz, (2, 0, 1))
    centers_flat = _fps_centers(planes[0], planes[1], planes[2])
    center = centers_flat.reshape(B, NUM_GROUP, 3)
    pts_flat, idx_flat = _knn_groups(planes[0], planes[1], planes[2],
                                     centers_flat)
    pts48 = pts_flat.reshape(B, NUM_GROUP, NCAND, 3)
    idx48 = idx_flat.reshape(B, NUM_GROUP, NCAND)
    d48 = (jnp.sum(center ** 2, -1)[:, :, None]
           + jnp.sum(pts48 ** 2, -1)
           - 2.0 * jnp.einsum('bgd,bgkd->bgk', center, pts48))
    posv = jnp.broadcast_to(jnp.arange(NCAND, dtype=jnp.int32),
                            idx48.shape)
    _, _, pos_sorted = jax.lax.sort((d48, idx48, posv), num_keys=2,
                                    dimension=2)
    pos32 = pos_sorted[:, :, :GROUP_SIZE]
    sel = jnp.take_along_axis(pts48, pos32[..., None], axis=2)
    neighborhood = sel - center[:, :, None, :]
    return (neighborhood, center)
